# baseline (speedup 1.0000x reference)
import jax
import jax.numpy as jnp
from jax.experimental import pallas as pl
from jax.experimental.pallas import tpu as pltpu

_CH = 200
_NBUF = 5
_LOOK = 4


def _chunk_copy(adj_hbm, buf, sems, chunk, slot):
    return pltpu.make_async_copy(
        adj_hbm.at[pl.ds(chunk * _CH, _CH), :],
        buf.at[slot],
        sems.at[slot],
    )


def _sage_kernel(adj_hbm, featb_ref, w1t_ref, w2t_ref, out_ref, buf, sems):
    i = pl.program_id(0)
    nsteps = pl.num_programs(0)

    @pl.when(i == 0)
    def _prologue():
        for k in range(_LOOK):
            _chunk_copy(adj_hbm, buf, sems, k, k).start()

    @pl.when(i + _LOOK < nsteps)
    def _issue_ahead():
        nxt = i + _LOOK
        _chunk_copy(adj_hbm, buf, sems, nxt, jax.lax.rem(nxt, _NBUF)).start()

    slot = jax.lax.rem(i, _NBUF)
    _chunk_copy(adj_hbm, buf, sems, i, slot).wait()

    a = buf[slot]
    ab = a.astype(jnp.bfloat16)
    fb = featb_ref[...]
    acc = jnp.dot(ab, fb, preferred_element_type=jnp.float32)
    n = a.shape[1]
    nfull = (n // 128) * 128
    part = a[:, 0:128]
    for c in range(1, nfull // 128):
        part = part + a[:, c * 128:(c + 1) * 128]
    rs = jnp.sum(part, axis=1, keepdims=True)
    if nfull < n:
        rs = rs + jnp.sum(a[:, nfull:n], axis=1, keepdims=True)
    neigh = acc / (rs + 1.0)
    f_blk = featb_ref[pl.ds(i * _CH, _CH), :]
    self_term = jnp.dot(f_blk, w1t_ref[...],
                        preferred_element_type=jnp.float32)
    neigh_term = jnp.dot(neigh.astype(jnp.bfloat16), w2t_ref[...],
                         preferred_element_type=jnp.float32)
    out_ref[...] = self_term + neigh_term


def kernel(adj, features, W):
    n = adj.shape[0]
    d = features.shape[1]
    d_out = W.shape[0]
    w1t = W[:, :d].T.astype(jnp.bfloat16)
    w2t = W[:, d:].T.astype(jnp.bfloat16)
    featb = features.astype(jnp.bfloat16)
    return pl.pallas_call(
        _sage_kernel,
        grid=(n // _CH,),
        in_specs=[
            pl.BlockSpec(memory_space=pltpu.MemorySpace.HBM),
            pl.BlockSpec((n, d), lambda i: (0, 0)),
            pl.BlockSpec((d, d_out), lambda i: (0, 0)),
            pl.BlockSpec((d, d_out), lambda i: (0, 0)),
        ],
        out_specs=pl.BlockSpec((_CH, d_out), lambda i: (i, 0)),
        out_shape=jax.ShapeDtypeStruct((n, d_out), jnp.float32),
        scratch_shapes=[
            pltpu.VMEM((_NBUF, _CH, n), jnp.float32),
            pltpu.SemaphoreType.DMA((_NBUF,)),
        ],
    )(adj, featb, w1t, w2t)


# speedup vs baseline: 1.0271x; 1.0271x over previous
import jax
import jax.numpy as jnp
from jax.experimental import pallas as pl

_BM = 400
_HB = 200


def _sage_kernel(a1_ref, a2_ref, featb_ref, w1t_ref, w2t_ref, out_ref):
    i = pl.program_id(0)
    fb = featb_ref[...]
    for j, ar in enumerate((a1_ref, a2_ref)):
        a = ar[...]
        ab = a.astype(jnp.bfloat16)
        acc = jnp.dot(ab, fb, preferred_element_type=jnp.float32)
        n = a.shape[1]
        nfull = (n // 128) * 128
        part = a[:, 0:128]
        for c in range(1, nfull // 128):
            part = part + a[:, c * 128:(c + 1) * 128]
        rs = jnp.sum(part, axis=1, keepdims=True)
        if nfull < n:
            rs = rs + jnp.sum(a[:, nfull:n], axis=1, keepdims=True)
        neigh = acc / (rs + 1.0)
        f_blk = featb_ref[pl.ds(i * _BM + j * _HB, _HB), :]
        self_term = jnp.dot(f_blk, w1t_ref[...],
                            preferred_element_type=jnp.float32)
        neigh_term = jnp.dot(neigh.astype(jnp.bfloat16), w2t_ref[...],
                             preferred_element_type=jnp.float32)
        out_ref[j * _HB:(j + 1) * _HB, :] = self_term + neigh_term


def kernel(adj, features, W):
    n = adj.shape[0]
    d = features.shape[1]
    d_out = W.shape[0]
    w1t = W[:, :d].T.astype(jnp.bfloat16)
    w2t = W[:, d:].T.astype(jnp.bfloat16)
    featb = features.astype(jnp.bfloat16)
    return pl.pallas_call(
        _sage_kernel,
        grid=(n // _BM,),
        in_specs=[
            pl.BlockSpec((_HB, n), lambda i: (2 * i, 0)),
            pl.BlockSpec((_HB, n), lambda i: (2 * i + 1, 0)),
            pl.BlockSpec((n, d), lambda i: (0, 0)),
            pl.BlockSpec((d, d_out), lambda i: (0, 0)),
            pl.BlockSpec((d, d_out), lambda i: (0, 0)),
        ],
        out_specs=pl.BlockSpec((_BM, d_out), lambda i: (i, 0)),
        out_shape=jax.ShapeDtypeStruct((n, d_out), jnp.float32),
    )(adj, adj, featb, w1t, w2t)
